# baseline (speedup 1.0000x reference)
import functools

import jax
import jax.numpy as jnp
from jax import lax
from jax.experimental import pallas as pl
from jax.experimental.pallas import tpu as pltpu
from jax.experimental.pallas import tpu_sc as plsc

N = 10000
E = 160000
CW = 128
NUM_CORES = 2
NUM_SUBCORES = 16
EDGE_BLOCK = 112
EPT = E // NUM_SUBCORES
N_BLOCKS = -(-EPT // EDGE_BLOCK)
EPT_PAD = N_BLOCKS * EDGE_BLOCK
N_ACC = N + 8
IDX_STAGE = 24
NSLOTS = 3


def _make_agg(n_chunks):
  chunks_per_core = n_chunks // NUM_CORES
  rows_per_tile = 624
  rows_rem = N - rows_per_tile * NUM_SUBCORES
  stages = []
  s0 = 0
  while s0 < N_BLOCKS:
    stages.append((s0, min(IDX_STAGE, N_BLOCKS - s0)))
    s0 += IDX_STAGE
  assert all(nb % NSLOTS == 0 for _, nb in stages)

  mesh = plsc.VectorSubcoreMesh(core_axis_name="c", subcore_axis_name="s")

  @functools.partial(
      pl.kernel,
      out_type=jax.ShapeDtypeStruct((n_chunks, N, CW), jnp.float32),
      mesh=mesh,
      scratch_types=[
          pltpu.VMEM((IDX_STAGE, EDGE_BLOCK), jnp.int32),
          pltpu.VMEM((IDX_STAGE, EDGE_BLOCK), jnp.int32),
          [pltpu.VMEM((EDGE_BLOCK, CW), jnp.float32)] * NSLOTS,
          [pltpu.SemaphoreType.DMA] * NSLOTS,
          [pltpu.SemaphoreType.DMA] * NSLOTS,
          pltpu.VMEM_SHARED((N_ACC, CW), jnp.float32),
      ],
  )
  def agg(h_hbm, src_hbm, dst_hbm, out_hbm,
          src_v, dst_v, rows_v, gsem, ssem, acc_sh):
    cid = lax.axis_index("c")
    sid = lax.axis_index("s")
    row0 = sid * rows_per_tile

    for local_c in range(chunks_per_core):
      c = cid * chunks_per_core + local_c
      pltpu.sync_copy(h_hbm.at[c, pl.ds(row0, rows_per_tile)],
                      acc_sh.at[pl.ds(row0, rows_per_tile)])

      @pl.when(sid == 0)
      def _(c=c):
        pltpu.sync_copy(h_hbm.at[c, pl.ds(N - rows_rem, rows_rem)],
                        acc_sh.at[pl.ds(N - rows_rem, rows_rem)])

      plsc.subcore_barrier()

      def gather(blk, slot, c=c):
        pltpu.async_copy(h_hbm.at[c].at[src_v.at[blk]], rows_v[slot],
                         gsem[slot])

      def wait_gather(blk, slot, c=c):
        pltpu.make_async_copy(h_hbm.at[c].at[src_v.at[blk]], rows_v[slot],
                              gsem[slot]).wait()

      def scatter(blk, slot):
        pltpu.async_copy(rows_v[slot], acc_sh.at[dst_v.at[blk]], ssem[slot],
                         add=True)

      def wait_scatter(blk, slot):
        pltpu.make_async_copy(rows_v[slot], acc_sh.at[dst_v.at[blk]],
                              ssem[slot]).wait()

      for s0, snb in stages:
        pltpu.sync_copy(src_hbm.at[sid, pl.ds(s0, snb)],
                        src_v.at[pl.ds(0, snb)])
        pltpu.sync_copy(dst_hbm.at[sid, pl.ds(s0, snb)],
                        dst_v.at[pl.ds(0, snb)])
        gather(0, 0)
        gather(1, 1)

        def body(q, _, snb=snb):
          for t in range(NSLOTS):
            blk = NSLOTS * q + t
            wait_gather(blk, t)
            scatter(blk, t)
            nxt = (t + 2) % NSLOTS

            @pl.when(blk + 2 < snb)
            def _(blk=blk, nxt=nxt):
              @pl.when(blk >= 1)
              def _():
                wait_scatter(jnp.maximum(blk - 1, 0), nxt)

              gather(blk + 2, nxt)

          return _

        lax.fori_loop(0, snb // NSLOTS, body, None)
        for k in range(min(NSLOTS, snb)):
          blkw = snb - NSLOTS + k
          wait_scatter(blkw, blkw % NSLOTS)

      plsc.subcore_barrier()
      pltpu.sync_copy(acc_sh.at[pl.ds(row0, rows_per_tile)],
                      out_hbm.at[c, pl.ds(row0, rows_per_tile)])

      @pl.when(sid == 0)
      def _(c=c):
        pltpu.sync_copy(acc_sh.at[pl.ds(N - rows_rem, rows_rem)],
                        out_hbm.at[c, pl.ds(N - rows_rem, rows_rem)])

      if local_c + 1 < chunks_per_core:
        plsc.subcore_barrier()

  return agg


_agg_l1 = _make_agg(2)
_agg_l2 = _make_agg(8)

_MB = 1000


def _m1_body(s_ref, w_ref, b_ref, o_ref):
  z = jnp.dot(s_ref[0].astype(jnp.bfloat16), w_ref[0:CW, :],
              preferred_element_type=jnp.float32)
  z += jnp.dot(s_ref[1].astype(jnp.bfloat16), w_ref[CW:2 * CW, :],
               preferred_element_type=jnp.float32)
  h = jnp.maximum(z + b_ref[...], 0.0)
  for j in range(8):
    o_ref[j] = h[:, j * CW:(j + 1) * CW]


def _layer1_matmul(s1, w1, b1):
  return pl.pallas_call(
      _m1_body,
      grid=(N // _MB,),
      in_specs=[
          pl.BlockSpec((2, _MB, CW), lambda i: (0, i, 0)),
          pl.BlockSpec((2 * CW, 8 * CW), lambda i: (0, 0)),
          pl.BlockSpec((1, 8 * CW), lambda i: (0, 0)),
      ],
      out_specs=pl.BlockSpec((8, _MB, CW), lambda i: (0, i, 0)),
      out_shape=jax.ShapeDtypeStruct((8, N, CW), jnp.float32),
  )(s1, w1, b1)


def _m2_body(s_ref, w2_ref, b2_ref, wf_ref, bf_ref, o_ref, acc_ref):
  i = pl.program_id(0)
  z = jnp.dot(s_ref[0].astype(jnp.bfloat16), w2_ref[0:CW, :],
              preferred_element_type=jnp.float32)
  for c in range(1, 8):
    z += jnp.dot(s_ref[c].astype(jnp.bfloat16), w2_ref[c * CW:(c + 1) * CW, :],
                 preferred_element_type=jnp.float32)
  h = jnp.maximum(z + b2_ref[...], 0.0)
  cs = jnp.sum(h, axis=0, keepdims=True)

  @pl.when(i == 0)
  def _():
    acc_ref[...] = cs

  @pl.when(i > 0)
  def _():
    acc_ref[...] += cs

  @pl.when(i == N // _MB - 1)
  def _():
    o_ref[...] = (jnp.dot(acc_ref[...], wf_ref[...],
                          preferred_element_type=jnp.float32)
                  + jnp.float32(N) * bf_ref[...])


def _layer2_head(s2, w2, b2, wf, bf):
  return pl.pallas_call(
      _m2_body,
      grid=(N // _MB,),
      in_specs=[
          pl.BlockSpec((8, _MB, CW), lambda i: (0, i, 0)),
          pl.BlockSpec((8 * CW, 8 * CW), lambda i: (0, 0)),
          pl.BlockSpec((1, 8 * CW), lambda i: (0, 0)),
          pl.BlockSpec((8 * CW, 2 * CW), lambda i: (0, 0)),
          pl.BlockSpec((1, 2 * CW), lambda i: (0, 0)),
      ],
      out_specs=pl.BlockSpec((1, 2 * CW), lambda i: (0, 0)),
      out_shape=jax.ShapeDtypeStruct((1, 2 * CW), jnp.float32),
      scratch_shapes=[pltpu.VMEM((1, 8 * CW), jnp.float32)],
  )(s2, w2, b2, wf, bf)


@jax.jit
def kernel(x, edge_index, W1, b1, W2, b2, Wf, bf):
  pad = ((0, 0), (0, EPT_PAD - EPT))
  src = jnp.pad(edge_index[0].reshape(NUM_SUBCORES, EPT), pad,
                constant_values=0)
  src = src.reshape(NUM_SUBCORES, N_BLOCKS, EDGE_BLOCK)
  dst = jnp.pad(edge_index[1].reshape(NUM_SUBCORES, EPT), pad,
                constant_values=N)
  dst = dst.reshape(NUM_SUBCORES, N_BLOCKS, EDGE_BLOCK)
  x3 = x.reshape(N, 2, CW).transpose(1, 0, 2)
  s1 = _agg_l1(x3, src, dst)
  h1 = _layer1_matmul(s1, W1.astype(jnp.bfloat16),
                      b1.reshape(1, -1))
  s2 = _agg_l2(h1, src, dst)
  return _layer2_head(s2, W2.astype(jnp.bfloat16), b2.reshape(1, -1),
                      Wf, bf.reshape(1, -1))


# speedup vs baseline: 1.0188x; 1.0188x over previous
import functools

import jax
import jax.numpy as jnp
from jax import lax
from jax.experimental import pallas as pl
from jax.experimental.pallas import tpu as pltpu
from jax.experimental.pallas import tpu_sc as plsc

N = 10000
E = 160000
CW = 128
NUM_CORES = 2
NUM_SUBCORES = 16
EDGE_BLOCK = 96
EPT = E // NUM_SUBCORES
N_BLOCKS = -(-EPT // EDGE_BLOCK)
EPT_PAD = N_BLOCKS * EDGE_BLOCK
N_ACC = N + 8
IDX_STAGE = 24
NSLOTS = 3


def _make_agg(n_chunks):
  chunks_per_core = n_chunks // NUM_CORES
  rows_per_tile = 624
  rows_rem = N - rows_per_tile * NUM_SUBCORES
  stages = []
  s0 = 0
  while s0 < N_BLOCKS:
    stages.append((s0, min(IDX_STAGE, N_BLOCKS - s0)))
    s0 += IDX_STAGE
  assert all(nb % NSLOTS == 0 for _, nb in stages)

  mesh = plsc.VectorSubcoreMesh(core_axis_name="c", subcore_axis_name="s")

  @functools.partial(
      pl.kernel,
      out_type=jax.ShapeDtypeStruct((n_chunks, N, CW), jnp.float32),
      mesh=mesh,
      scratch_types=[
          [pltpu.VMEM((IDX_STAGE, EDGE_BLOCK), jnp.int32)] * 2,
          [pltpu.VMEM((IDX_STAGE, EDGE_BLOCK), jnp.int32)] * 2,
          [pltpu.VMEM((EDGE_BLOCK, CW), jnp.float32)] * NSLOTS,
          [pltpu.SemaphoreType.DMA] * NSLOTS,
          [pltpu.SemaphoreType.DMA] * NSLOTS,
          pltpu.VMEM_SHARED((N_ACC, CW), jnp.float32),
      ],
  )
  def agg(h_hbm, src_hbm, dst_hbm, out_hbm,
          src_v, dst_v, rows_v, gsem, ssem, acc_sh):
    cid = lax.axis_index("c")
    sid = lax.axis_index("s")
    row0 = sid * rows_per_tile

    def seed(c):
      hc = h_hbm.at[c]
      pltpu.sync_copy(hc.at[pl.ds(row0, rows_per_tile)],
                      acc_sh.at[pl.ds(row0, rows_per_tile)])

      @pl.when(sid == 0)
      def _():
        pltpu.sync_copy(hc.at[pl.ds(N - rows_rem, rows_rem)],
                        acc_sh.at[pl.ds(N - rows_rem, rows_rem)])

    def writeout(c):
      oc = out_hbm.at[c]
      pltpu.sync_copy(acc_sh.at[pl.ds(row0, rows_per_tile)],
                      oc.at[pl.ds(row0, rows_per_tile)])

      @pl.when(sid == 0)
      def _():
        pltpu.sync_copy(acc_sh.at[pl.ds(N - rows_rem, rows_rem)],
                        oc.at[pl.ds(N - rows_rem, rows_rem)])

    def load_idx(s0, snb, p):
      pltpu.sync_copy(src_hbm.at[sid, pl.ds(s0, snb)],
                      src_v[p].at[pl.ds(0, snb)])
      pltpu.sync_copy(dst_hbm.at[sid, pl.ds(s0, snb)],
                      dst_v[p].at[pl.ds(0, snb)])

    def gather(c, blk, slot, p):
      pltpu.async_copy(h_hbm.at[c].at[src_v[p].at[blk]], rows_v[slot],
                       gsem[slot])

    def wait_gather(c, blk, slot, p):
      pltpu.make_async_copy(h_hbm.at[c].at[src_v[p].at[blk]], rows_v[slot],
                            gsem[slot]).wait()

    def scatter(blk, slot, p):
      pltpu.async_copy(rows_v[slot], acc_sh.at[dst_v[p].at[blk]], ssem[slot],
                       add=True)

    def wait_scatter(blk, slot, p):
      pltpu.make_async_copy(rows_v[slot], acc_sh.at[dst_v[p].at[blk]],
                            ssem[slot]).wait()

    segments = []
    for local_c in range(chunks_per_core):
      for s0, snb in stages:
        segments.append((local_c, s0, snb))

    c0 = cid * chunks_per_core
    seed(c0)
    plsc.subcore_barrier()
    load_idx(stages[0][0], stages[0][1], 0)
    gather(c0, 0, 0, 0)
    gather(c0, 1, 1, 0)

    for gi, (local_c, s0, snb) in enumerate(segments):
      c = c0 + local_c
      p = gi % 2
      nxt_seg = segments[gi + 1] if gi + 1 < len(segments) else None
      if nxt_seg is not None:
        load_idx(nxt_seg[1], nxt_seg[2], 1 - p)

      def body(q, _, c=c, snb=snb, p=p):
        for t in range(NSLOTS):
          blk = NSLOTS * q + t
          wait_gather(c, blk, t, p)
          scatter(blk, t, p)
          nxt = (t + 2) % NSLOTS

          @pl.when(blk + 2 < snb)
          def _(blk=blk, nxt=nxt):
            @pl.when(blk >= 1)
            def _():
              wait_scatter(jnp.maximum(blk - 1, 0), nxt, p)

            gather(c, blk + 2, nxt, p)

        return _

      lax.fori_loop(0, snb // NSLOTS, body, None)
      for k in range(NSLOTS):
        blkw = snb - NSLOTS + k
        wait_scatter(blkw, blkw % NSLOTS, p)

      if nxt_seg is not None and nxt_seg[0] == local_c:
        gather(c, 0, 0, 1 - p)
        gather(c, 1, 1, 1 - p)
      elif nxt_seg is not None:
        c_n = c0 + nxt_seg[0]
        gather(c_n, 0, 0, 1 - p)
        gather(c_n, 1, 1, 1 - p)
        plsc.subcore_barrier()
        writeout(c)
        seed(c_n)
        plsc.subcore_barrier()
      else:
        plsc.subcore_barrier()
        writeout(c)

  return agg


_agg_l1 = _make_agg(2)
_agg_l2 = _make_agg(8)

_MB = 1000


def _m1_body(s_ref, w_ref, b_ref, o_ref):
  z = jnp.dot(s_ref[0], w_ref[0:CW, :], preferred_element_type=jnp.float32)
  z += jnp.dot(s_ref[1], w_ref[CW:2 * CW, :], preferred_element_type=jnp.float32)
  h = jnp.maximum(z + b_ref[...], 0.0)
  for j in range(8):
    o_ref[j] = h[:, j * CW:(j + 1) * CW]


def _layer1_matmul(s1, w1, b1):
  return pl.pallas_call(
      _m1_body,
      grid=(N // _MB,),
      in_specs=[
          pl.BlockSpec((2, _MB, CW), lambda i: (0, i, 0)),
          pl.BlockSpec((2 * CW, 8 * CW), lambda i: (0, 0)),
          pl.BlockSpec((1, 8 * CW), lambda i: (0, 0)),
      ],
      out_specs=pl.BlockSpec((8, _MB, CW), lambda i: (0, i, 0)),
      out_shape=jax.ShapeDtypeStruct((8, N, CW), jnp.float32),
  )(s1, w1, b1)


def _m2_body(s_ref, w2_ref, b2_ref, wf_ref, bf_ref, o_ref, acc_ref):
  i = pl.program_id(0)
  z = jnp.dot(s_ref[0], w2_ref[0:CW, :], preferred_element_type=jnp.float32)
  for c in range(1, 8):
    z += jnp.dot(s_ref[c], w2_ref[c * CW:(c + 1) * CW, :],
                 preferred_element_type=jnp.float32)
  h = jnp.maximum(z + b2_ref[...], 0.0)
  cs = jnp.sum(h, axis=0, keepdims=True)

  @pl.when(i == 0)
  def _():
    acc_ref[...] = cs

  @pl.when(i > 0)
  def _():
    acc_ref[...] += cs

  @pl.when(i == N // _MB - 1)
  def _():
    o_ref[...] = (jnp.dot(acc_ref[...], wf_ref[...],
                          preferred_element_type=jnp.float32)
                  + jnp.float32(N) * bf_ref[...])


def _layer2_head(s2, w2, b2, wf, bf):
  return pl.pallas_call(
      _m2_body,
      grid=(N // _MB,),
      in_specs=[
          pl.BlockSpec((8, _MB, CW), lambda i: (0, i, 0)),
          pl.BlockSpec((8 * CW, 8 * CW), lambda i: (0, 0)),
          pl.BlockSpec((1, 8 * CW), lambda i: (0, 0)),
          pl.BlockSpec((8 * CW, 2 * CW), lambda i: (0, 0)),
          pl.BlockSpec((1, 2 * CW), lambda i: (0, 0)),
      ],
      out_specs=pl.BlockSpec((1, 2 * CW), lambda i: (0, 0)),
      out_shape=jax.ShapeDtypeStruct((1, 2 * CW), jnp.float32),
      scratch_shapes=[pltpu.VMEM((1, 8 * CW), jnp.float32)],
  )(s2, w2, b2, wf, bf)


@jax.jit
def kernel(x, edge_index, W1, b1, W2, b2, Wf, bf):
  pad = ((0, 0), (0, EPT_PAD - EPT))
  src = jnp.pad(edge_index[0].reshape(NUM_SUBCORES, EPT), pad,
                constant_values=0)
  src = src.reshape(NUM_SUBCORES, N_BLOCKS, EDGE_BLOCK)
  dst = jnp.pad(edge_index[1].reshape(NUM_SUBCORES, EPT), pad,
                constant_values=N)
  dst = dst.reshape(NUM_SUBCORES, N_BLOCKS, EDGE_BLOCK)
  x3 = x.reshape(N, 2, CW).transpose(1, 0, 2)
  s1 = _agg_l1(x3, src, dst)
  h1 = _layer1_matmul(s1, W1, b1.reshape(1, -1))
  s2 = _agg_l2(h1, src, dst)
  return _layer2_head(s2, W2, b2.reshape(1, -1), Wf, bf.reshape(1, -1))
